# Initial kernel scaffold; baseline (speedup 1.0000x reference)
import jax
import jax.numpy as jnp
from jax.experimental import pallas as pl


def kernel(verts, edges, subdivided_faces, W1, b1, W2, b2, W3, b3):
    raise NotImplementedError("write your pallas kernel here")


# speedup vs baseline: 8.4233x; 8.4233x over previous
import functools

import jax
import jax.numpy as jnp
from jax import lax
from jax.experimental import pallas as pl
from jax.experimental.pallas import tpu as pltpu
from jax.experimental.pallas import tpu_sc as plsc

N = 50000
E = 800000
NP = 51200
EP = 819200
NW = 32
CH = 128
CPT = EP // NW // CH
EPT = EP // NW
RPS = NP // 16
DUMMY = 50100
BLK = 1024
GRID_N = NP // BLK

_mesh = plsc.VectorSubcoreMesh(core_axis_name="c", subcore_axis_name="s")
_sc_params = pltpu.CompilerParams(use_tc_tiling_on_sc=False)


def _sc_scatter(w):
  @functools.partial(
      pl.kernel,
      out_type=jax.ShapeDtypeStruct((2, NP, w), jnp.float32),
      mesh=_mesh,
      compiler_params=_sc_params,
      scratch_types=[
          pltpu.VMEM((CH,), jnp.int32),
          pltpu.VMEM((CH,), jnp.int32),
          pltpu.VMEM((CH, w), jnp.float32),
          pltpu.VMEM_SHARED((NP, w), jnp.float32),
          pltpu.SemaphoreType.DMA,
      ],
  )
  def k(src_hbm, dst_hbm, y_hbm, z_hbm, out_hbm, src_v, dst_v, rows_v, acc_sh, sem):
    c = lax.axis_index("c")
    s = lax.axis_index("s")
    wid = s * 2 + c
    pltpu.sync_copy(z_hbm.at[pl.ds(s * RPS, RPS)], acc_sh.at[pl.ds(s * RPS, RPS)])
    plsc.subcore_barrier()
    base0 = wid * EPT

    def body(i, carry):
      base = base0 + i * CH
      pltpu.sync_copy(src_hbm.at[pl.ds(base, CH)], src_v)
      pltpu.async_copy(y_hbm.at[src_v], rows_v, sem).wait()
      pltpu.sync_copy(dst_hbm.at[pl.ds(base, CH)], dst_v)
      pltpu.sync_copy(rows_v, acc_sh.at[dst_v], add=True)
      return carry

    lax.fori_loop(0, CPT, body, 0)
    plsc.subcore_barrier()
    pltpu.sync_copy(acc_sh.at[pl.ds(s * RPS, RPS)],
                    out_hbm.at[c].at[pl.ds(s * RPS, RPS)])

  return k


@functools.partial(
    pl.kernel,
    out_type=jax.ShapeDtypeStruct((2, NP), jnp.float32),
    mesh=_mesh,
    compiler_params=_sc_params,
    scratch_types=[
        pltpu.VMEM((CH,), jnp.int32),
        pltpu.VMEM((CH,), jnp.float32),
        pltpu.VMEM_SHARED((NP,), jnp.float32),
        pltpu.SemaphoreType.DMA,
    ],
)
def _sc_degree(dst_hbm, z_hbm, out_hbm, dst_v, ones_v, acc_sh, sem):
  c = lax.axis_index("c")
  s = lax.axis_index("s")
  wid = s * 2 + c
  one = jnp.full((16,), 1.0, jnp.float32)
  for j in range(CH // 16):
    ones_v[pl.ds(j * 16, 16)] = one
  pltpu.sync_copy(z_hbm.at[pl.ds(s * RPS, RPS)], acc_sh.at[pl.ds(s * RPS, RPS)])
  plsc.subcore_barrier()
  base0 = wid * EPT

  def body(i, carry):
    base = base0 + i * CH
    pltpu.sync_copy(dst_hbm.at[pl.ds(base, CH)], dst_v)
    pltpu.sync_copy(ones_v, acc_sh.at[dst_v], add=True)
    return carry

  lax.fori_loop(0, CPT, body, 0)
  plsc.subcore_barrier()
  pltpu.sync_copy(acc_sh.at[pl.ds(s * RPS, RPS)],
                  out_hbm.at[c].at[pl.ds(s * RPS, RPS)])


@functools.partial(
    pl.kernel,
    out_type=[jax.ShapeDtypeStruct((EP, 8), jnp.float32),
              jax.ShapeDtypeStruct((EP, 8), jnp.float32)],
    mesh=_mesh,
    compiler_params=_sc_params,
    scratch_types=[
        pltpu.VMEM((CH,), jnp.int32),
        pltpu.VMEM((CH, 8), jnp.float32),
        pltpu.SemaphoreType.DMA,
    ],
)
def _sc_midgather(src_hbm, dst_hbm, vh_hbm, gs_hbm, gd_hbm, idx_v, rows_v, sem):
  c = lax.axis_index("c")
  s = lax.axis_index("s")
  wid = s * 2 + c
  base0 = wid * EPT

  def body(i, carry):
    base = base0 + i * CH
    pltpu.sync_copy(src_hbm.at[pl.ds(base, CH)], idx_v)
    pltpu.async_copy(vh_hbm.at[idx_v], rows_v, sem).wait()
    pltpu.sync_copy(rows_v, gs_hbm.at[pl.ds(base, CH)])
    pltpu.sync_copy(dst_hbm.at[pl.ds(base, CH)], idx_v)
    pltpu.async_copy(vh_hbm.at[idx_v], rows_v, sem).wait()
    pltpu.sync_copy(rows_v, gd_hbm.at[pl.ds(base, CH)])
    return carry

  lax.fori_loop(0, CPT, body, 0)


def _leaky(x):
  return jnp.where(x >= 0, x, 0.01 * x)


def _row_spec(w):
  if w == 1:
    return pl.BlockSpec((BLK,), lambda i: (i,))
  return pl.BlockSpec((BLK, w), lambda i: (i, 0))


def _full_spec(shape):
  nd = len(shape)
  return pl.BlockSpec(shape, lambda i: (0,) * nd)


def _tc0_body(d0, d1, v4, dis, y1):
  deg = d0[...] + d1[...] + 1.0
  r = lax.rsqrt(deg)
  dis[...] = r
  y1[...] = v4[...] * r[:, None]


def _tc1_body(a0, a1, y1, dis, w1, b1, w2, y2):
  t = (a0[...] + a1[...] + y1[...]) * dis[...][:, None]
  x1 = _leaky(jnp.dot(t[:, :3], w1[...], preferred_element_type=jnp.float32)
              + b1[...][None, :])
  h1 = jnp.dot(x1, w2[...], preferred_element_type=jnp.float32)
  y2[...] = h1 * dis[...][:, None]


def _tc2_body(a0, a1, y2, dis, b2, w3p, y3p):
  x2 = _leaky((a0[...] + a1[...] + y2[...]) * dis[...][:, None] + b2[...][None, :])
  h2 = jnp.dot(x2, w3p[...], preferred_element_type=jnp.float32)
  y3p[...] = h2 * dis[...][:, None]


def _tc3_body(a0, a1, y3p, dis, v4, b3p, vout, vh):
  off = (a0[...] + a1[...] + y3p[...]) * dis[...][:, None] + b3p[...][None, :]
  v = v4[...] + off
  vout[...] = v
  vh[...] = 0.5 * v


def _tc4_body(gs, gd, mid):
  mid[...] = gs[...] + gd[...]


_scatter8 = _sc_scatter(8)
_scatter32 = _sc_scatter(32)

_tc0 = pl.pallas_call(
    _tc0_body, grid=(GRID_N,),
    in_specs=[_row_spec(1), _row_spec(1), _row_spec(8)],
    out_specs=[_row_spec(1), _row_spec(8)],
    out_shape=[jax.ShapeDtypeStruct((NP,), jnp.float32),
               jax.ShapeDtypeStruct((NP, 8), jnp.float32)],
)

_tc1 = pl.pallas_call(
    _tc1_body, grid=(GRID_N,),
    in_specs=[_row_spec(8), _row_spec(8), _row_spec(8), _row_spec(1),
              _full_spec((3, 64)), _full_spec((64,)), _full_spec((64, 32))],
    out_specs=_row_spec(32),
    out_shape=jax.ShapeDtypeStruct((NP, 32), jnp.float32),
)

_tc2 = pl.pallas_call(
    _tc2_body, grid=(GRID_N,),
    in_specs=[_row_spec(32), _row_spec(32), _row_spec(32), _row_spec(1),
              _full_spec((32,)), _full_spec((32, 8))],
    out_specs=_row_spec(8),
    out_shape=jax.ShapeDtypeStruct((NP, 8), jnp.float32),
)

_tc3 = pl.pallas_call(
    _tc3_body, grid=(GRID_N,),
    in_specs=[_row_spec(8), _row_spec(8), _row_spec(8), _row_spec(1),
              _row_spec(8), _full_spec((8,))],
    out_specs=[_row_spec(8), _row_spec(8)],
    out_shape=[jax.ShapeDtypeStruct((NP, 8), jnp.float32),
               jax.ShapeDtypeStruct((NP, 8), jnp.float32)],
)

_tc4 = pl.pallas_call(
    _tc4_body, grid=(EP // BLK,),
    in_specs=[_row_spec(8), _row_spec(8)],
    out_specs=_row_spec(8),
    out_shape=jax.ShapeDtypeStruct((EP, 8), jnp.float32),
)


def kernel(verts, edges, subdivided_faces, W1, b1, W2, b2, W3, b3):
  src = edges[:, 0]
  dst = edges[:, 1]
  pad_e = EP - E
  src_p = jnp.concatenate([src, jnp.zeros((pad_e,), jnp.int32)])
  dst_p = jnp.concatenate([dst, jnp.full((pad_e,), DUMMY, jnp.int32)])
  verts8 = jnp.pad(verts, ((0, NP - N), (0, 5)))
  w3p = jnp.pad(W3, ((0, 0), (0, 5)))
  b3p = jnp.pad(b3, (0, 5))
  z1 = jnp.zeros((NP,), jnp.float32)
  z8 = jnp.zeros((NP, 8), jnp.float32)
  z32 = jnp.zeros((NP, 32), jnp.float32)

  degp = _sc_degree(dst_p, z1)
  dis, y1 = _tc0(degp[0], degp[1], verts8)
  acc1 = _scatter8(src_p, dst_p, y1, z8)
  y2 = _tc1(acc1[0], acc1[1], y1, dis, W1, b1, W2)
  acc2 = _scatter32(src_p, dst_p, y2, z32)
  y3p = _tc2(acc2[0], acc2[1], y2, dis, b2, w3p)
  acc3 = _scatter8(src_p, dst_p, y3p, z8)
  v8, vh = _tc3(acc3[0], acc3[1], y3p, dis, verts8, b3p)
  gs, gd = _sc_midgather(src_p, dst_p, vh)
  mid8 = _tc4(gs, gd)

  new_verts = jnp.concatenate([v8[:N, :3], mid8[:E, :3]], axis=0)[None]
  new_faces = subdivided_faces[None]
  return new_verts, new_faces


# speedup vs baseline: 11.7095x; 1.3901x over previous
import functools

import jax
import jax.numpy as jnp
from jax import lax
from jax.experimental import pallas as pl
from jax.experimental.pallas import tpu as pltpu
from jax.experimental.pallas import tpu_sc as plsc

N = 50000
E = 800000
NP = 51200
EP = 819200
NW = 32
CH = 128
CPT = EP // NW // CH
NCH = EP // CH
EPT = EP // NW
RPS = NP // 16
DUMMY = 50100
NB = 8
NGRP = CPT // NB
BLK = 1024
GRID_N = NP // BLK

_mesh = plsc.VectorSubcoreMesh(core_axis_name="c", subcore_axis_name="s")
_sc_params = pltpu.CompilerParams(use_tc_tiling_on_sc=False)


def _sc_scatter(w):
  @functools.partial(
      pl.kernel,
      out_type=jax.ShapeDtypeStruct((2, NP, w), jnp.float32),
      mesh=_mesh,
      compiler_params=_sc_params,
      scratch_types=[
          pltpu.VMEM((CPT, CH), jnp.int32),
          pltpu.VMEM((CPT, CH), jnp.int32),
          pltpu.VMEM((NB, CH, w), jnp.float32),
          pltpu.VMEM_SHARED((NP, w), jnp.float32),
          pltpu.SemaphoreType.DMA((NB,)),
          pltpu.SemaphoreType.DMA((NB,)),
      ],
  )
  def k(src_hbm, dst_hbm, y_hbm, z_hbm, out_hbm, sidx, didx, rows, acc_sh,
        gsem, ssem):
    c = lax.axis_index("c")
    s = lax.axis_index("s")
    wid = s * 2 + c
    pltpu.sync_copy(z_hbm.at[pl.ds(s * RPS, RPS)], acc_sh.at[pl.ds(s * RPS, RPS)])
    pltpu.sync_copy(src_hbm.at[pl.ds(wid * CPT, CPT)], sidx)
    pltpu.sync_copy(dst_hbm.at[pl.ds(wid * CPT, CPT)], didx)
    plsc.subcore_barrier()

    for b in range(NB):
      pltpu.async_copy(y_hbm.at[sidx.at[b]], rows.at[b], gsem.at[b])

    def outer(g, carry):
      base = g * NB
      for b in range(NB):
        pltpu.make_async_copy(y_hbm.at[sidx.at[base + b]], rows.at[b],
                              gsem.at[b]).wait()
        pltpu.async_copy(rows.at[b], acc_sh.at[didx.at[base + b]], ssem.at[b],
                         add=True)
      for b in range(NB):
        nxt = base + NB + b

        @pl.when(nxt < CPT)
        def _():
          pltpu.make_async_copy(rows.at[b], acc_sh.at[didx.at[base + b]],
                                ssem.at[b]).wait()
          pltpu.async_copy(y_hbm.at[sidx.at[nxt]], rows.at[b], gsem.at[b])

      return carry

    lax.fori_loop(0, NGRP, outer, 0)
    for b in range(NB):
      pltpu.make_async_copy(rows.at[b], acc_sh.at[didx.at[CPT - NB + b]],
                            ssem.at[b]).wait()
    plsc.subcore_barrier()
    pltpu.sync_copy(acc_sh.at[pl.ds(s * RPS, RPS)],
                    out_hbm.at[c].at[pl.ds(s * RPS, RPS)])

  return k


@functools.partial(
    pl.kernel,
    out_type=jax.ShapeDtypeStruct((2, NP), jnp.float32),
    mesh=_mesh,
    compiler_params=_sc_params,
    scratch_types=[
        pltpu.VMEM((CPT, CH), jnp.int32),
        pltpu.VMEM((CH,), jnp.float32),
        pltpu.VMEM_SHARED((NP,), jnp.float32),
        pltpu.SemaphoreType.DMA((NB,)),
    ],
)
def _sc_degree(dst_hbm, z_hbm, out_hbm, didx, ones_v, acc_sh, ssem):
  c = lax.axis_index("c")
  s = lax.axis_index("s")
  wid = s * 2 + c
  one = jnp.full((16,), 1.0, jnp.float32)
  for j in range(CH // 16):
    ones_v[pl.ds(j * 16, 16)] = one
  pltpu.sync_copy(z_hbm.at[pl.ds(s * RPS, RPS)], acc_sh.at[pl.ds(s * RPS, RPS)])
  pltpu.sync_copy(dst_hbm.at[pl.ds(wid * CPT, CPT)], didx)
  plsc.subcore_barrier()

  for b in range(NB):
    pltpu.async_copy(ones_v, acc_sh.at[didx.at[b]], ssem.at[b], add=True)

  def outer(g, carry):
    base = g * NB
    for b in range(NB):
      nxt = base + NB + b

      @pl.when(nxt < CPT)
      def _():
        pltpu.make_async_copy(ones_v, acc_sh.at[didx.at[base + b]],
                              ssem.at[b]).wait()
        pltpu.async_copy(ones_v, acc_sh.at[didx.at[nxt]], ssem.at[b], add=True)

    return carry

  lax.fori_loop(0, NGRP, outer, 0)
  for b in range(NB):
    pltpu.make_async_copy(ones_v, acc_sh.at[didx.at[CPT - NB + b]],
                          ssem.at[b]).wait()
  plsc.subcore_barrier()
  pltpu.sync_copy(acc_sh.at[pl.ds(s * RPS, RPS)],
                  out_hbm.at[c].at[pl.ds(s * RPS, RPS)])


@functools.partial(
    pl.kernel,
    out_type=jax.ShapeDtypeStruct((EP, 8), jnp.float32),
    mesh=_mesh,
    compiler_params=_sc_params,
    scratch_types=[
        pltpu.VMEM((CPT, CH), jnp.int32),
        pltpu.VMEM((NB, CH, 8), jnp.float32),
        pltpu.SemaphoreType.DMA((NB,)),
        pltpu.SemaphoreType.DMA((NB,)),
    ],
)
def _sc_gatherrows(idx_hbm, vh_hbm, out_hbm, sidx, rows, gsem, wsem):
  c = lax.axis_index("c")
  s = lax.axis_index("s")
  wid = s * 2 + c
  pltpu.sync_copy(idx_hbm.at[pl.ds(wid * CPT, CPT)], sidx)
  base0 = wid * EPT

  for b in range(NB):
    pltpu.async_copy(vh_hbm.at[sidx.at[b]], rows.at[b], gsem.at[b])

  def outer(g, carry):
    base = g * NB
    for b in range(NB):
      ob = base0 + (base + b) * CH
      pltpu.make_async_copy(vh_hbm.at[sidx.at[base + b]], rows.at[b],
                            gsem.at[b]).wait()
      pltpu.async_copy(rows.at[b], out_hbm.at[pl.ds(ob, CH)], wsem.at[b])
    for b in range(NB):
      nxt = base + NB + b
      ob = base0 + (base + b) * CH

      @pl.when(nxt < CPT)
      def _():
        pltpu.make_async_copy(rows.at[b], out_hbm.at[pl.ds(ob, CH)],
                              wsem.at[b]).wait()
        pltpu.async_copy(vh_hbm.at[sidx.at[nxt]], rows.at[b], gsem.at[b])

    return carry

  lax.fori_loop(0, NGRP, outer, 0)
  for b in range(NB):
    obl = base0 + (CPT - NB + b) * CH
    pltpu.make_async_copy(rows.at[b], out_hbm.at[pl.ds(obl, CH)],
                          wsem.at[b]).wait()


def _leaky(x):
  return jnp.where(x >= 0, x, 0.01 * x)


def _row_spec(w):
  if w == 1:
    return pl.BlockSpec((BLK,), lambda i: (i,))
  return pl.BlockSpec((BLK, w), lambda i: (i, 0))


def _full_spec(shape):
  nd = len(shape)
  return pl.BlockSpec(shape, lambda i: (0,) * nd)


def _tc0_body(d0, d1, v8, dis, y1):
  deg = d0[...] + d1[...] + 1.0
  r = lax.rsqrt(deg)
  dis[...] = r
  y1[...] = v8[...] * r[:, None]


def _tc1_body(a0, a1, y1, dis, w1, b1, w2, y2a, y2b):
  t = (a0[...] + a1[...] + y1[...]) * dis[...][:, None]
  x1 = _leaky(jnp.dot(t[:, :3], w1[...], preferred_element_type=jnp.float32)
              + b1[...][None, :])
  h1 = jnp.dot(x1, w2[...], preferred_element_type=jnp.float32)
  y2 = h1 * dis[...][:, None]
  y2a[...] = y2[:, :16]
  y2b[...] = y2[:, 16:]


def _tc2_body(a0a, a1a, a0b, a1b, y2a, y2b, dis, b2, w3a, w3b, y3p):
  d = dis[...][:, None]
  x2a = _leaky((a0a[...] + a1a[...] + y2a[...]) * d + b2[...][None, :16])
  x2b = _leaky((a0b[...] + a1b[...] + y2b[...]) * d + b2[...][None, 16:])
  h2 = (jnp.dot(x2a, w3a[...], preferred_element_type=jnp.float32)
        + jnp.dot(x2b, w3b[...], preferred_element_type=jnp.float32))
  y3p[...] = h2 * d


def _tc3_body(a0, a1, y3p, dis, v8, b3p, vout, vh):
  off = (a0[...] + a1[...] + y3p[...]) * dis[...][:, None] + b3p[...][None, :]
  v = v8[...] + off
  vout[...] = v
  vh[...] = 0.5 * v


def _tc4_body(gs, gd, mid):
  mid[...] = gs[...] + gd[...]


_scatter8 = _sc_scatter(8)
_scatter16 = _sc_scatter(16)

_tc0 = pl.pallas_call(
    _tc0_body, grid=(GRID_N,),
    in_specs=[_row_spec(1), _row_spec(1), _row_spec(8)],
    out_specs=[_row_spec(1), _row_spec(8)],
    out_shape=[jax.ShapeDtypeStruct((NP,), jnp.float32),
               jax.ShapeDtypeStruct((NP, 8), jnp.float32)],
)

_tc1 = pl.pallas_call(
    _tc1_body, grid=(GRID_N,),
    in_specs=[_row_spec(8), _row_spec(8), _row_spec(8), _row_spec(1),
              _full_spec((3, 64)), _full_spec((64,)), _full_spec((64, 32))],
    out_specs=[_row_spec(16), _row_spec(16)],
    out_shape=[jax.ShapeDtypeStruct((NP, 16), jnp.float32),
               jax.ShapeDtypeStruct((NP, 16), jnp.float32)],
)

_tc2 = pl.pallas_call(
    _tc2_body, grid=(GRID_N,),
    in_specs=[_row_spec(16), _row_spec(16), _row_spec(16), _row_spec(16),
              _row_spec(16), _row_spec(16), _row_spec(1),
              _full_spec((32,)), _full_spec((16, 8)), _full_spec((16, 8))],
    out_specs=_row_spec(8),
    out_shape=jax.ShapeDtypeStruct((NP, 8), jnp.float32),
)

_tc3 = pl.pallas_call(
    _tc3_body, grid=(GRID_N,),
    in_specs=[_row_spec(8), _row_spec(8), _row_spec(8), _row_spec(1),
              _row_spec(8), _full_spec((8,))],
    out_specs=[_row_spec(8), _row_spec(8)],
    out_shape=[jax.ShapeDtypeStruct((NP, 8), jnp.float32),
               jax.ShapeDtypeStruct((NP, 8), jnp.float32)],
)

_tc4 = pl.pallas_call(
    _tc4_body, grid=(EP // BLK,),
    in_specs=[_row_spec(8), _row_spec(8)],
    out_specs=_row_spec(8),
    out_shape=jax.ShapeDtypeStruct((EP, 8), jnp.float32),
)


def kernel(verts, edges, subdivided_faces, W1, b1, W2, b2, W3, b3):
  src = edges[:, 0]
  dst = edges[:, 1]
  pad_e = EP - E
  src_p = jnp.concatenate([src, jnp.zeros((pad_e,), jnp.int32)])
  dst_p = jnp.concatenate([dst, jnp.full((pad_e,), DUMMY, jnp.int32)])
  src2 = src_p.reshape(NCH, CH)
  dst2 = dst_p.reshape(NCH, CH)
  verts8 = jnp.pad(verts, ((0, NP - N), (0, 5)))
  w3p = jnp.pad(W3, ((0, 0), (0, 5)))
  w3a = w3p[:16]
  w3b = w3p[16:]
  b3p = jnp.pad(b3, (0, 5))
  z1 = jnp.zeros((NP,), jnp.float32)
  z8 = jnp.zeros((NP, 8), jnp.float32)
  z16 = jnp.zeros((NP, 16), jnp.float32)

  degp = _sc_degree(dst2, z1)
  dis, y1 = _tc0(degp[0], degp[1], verts8)
  acc1 = _scatter8(src2, dst2, y1, z8)
  y2a, y2b = _tc1(acc1[0], acc1[1], y1, dis, W1, b1, W2)
  acc2a = _scatter16(src2, dst2, y2a, z16)
  acc2b = _scatter16(src2, dst2, y2b, z16)
  y3p = _tc2(acc2a[0], acc2a[1], acc2b[0], acc2b[1], y2a, y2b, dis, b2, w3a, w3b)
  acc3 = _scatter8(src2, dst2, y3p, z8)
  v8, vh = _tc3(acc3[0], acc3[1], y3p, dis, verts8, b3p)
  gs = _sc_gatherrows(src2, vh)
  gd = _sc_gatherrows(dst2, vh)
  mid8 = _tc4(gs, gd)

  new_verts = jnp.concatenate([v8[:N, :3], mid8[:E, :3]], axis=0)[None]
  new_faces = subdivided_faces[None]
  return new_verts, new_faces
